# Initial kernel scaffold; baseline (speedup 1.0000x reference)
import jax
import jax.numpy as jnp
from jax.experimental import pallas as pl


def kernel(x):
    raise NotImplementedError("write your pallas kernel here")


# speedup vs baseline: 7.7329x; 7.7329x over previous
import numpy as np

import jax
import jax.numpy as jnp
from jax import lax
from jax.experimental import pallas as pl
from jax.experimental.pallas import tpu as pltpu
from jax.experimental.pallas import tpu_sc as plsc

TEMPERATURE = 2.0 / 3.0
K = 2048
ROWS = 64
COLS = 32768
L = 16
NV = COLS // L
NC = 2
NS = 16
HB = 4096
MIN32 = np.int32(-(2 ** 31))


def _find_bucket(hist_ref, nchunks, kk):
    lane = lax.broadcasted_iota(jnp.int32, (L,), 0)

    def body(j, carry):
        found, bsel, above, acc = carry
        c = nchunks - 1 - j
        h = hist_ref[pl.ds(c * L, L)]
        rev = lax.rev(h, (0,))
        cs = plsc.cumsum(rev)
        cum = cs + acc
        crossed = cum >= kk
        ncross = jnp.sum(crossed.astype(jnp.int32))
        any_crossed = ncross > 0
        t = L - ncross
        sel = lane == t
        above_here = jnp.sum(jnp.where(sel, cum - rev, 0))
        b_here = c * L + (L - 1 - t)
        is_here = jnp.logical_and(jnp.logical_not(found), any_crossed)
        bsel = jnp.where(is_here, b_here, bsel)
        above = jnp.where(is_here, above_here, above)
        found = jnp.logical_or(found, any_crossed)
        acc = acc + jnp.max(cs)
        return found, bsel, above, acc

    init = (jnp.bool_(False), jnp.int32(0), jnp.int32(0), jnp.int32(0))
    _, bsel, above, _ = lax.fori_loop(0, nchunks, body, init)
    return bsel, above


def _zero_hist(hist_ref, nchunks):
    zero = jnp.zeros((L,), jnp.int32)

    def body(i, _):
        hist_ref[pl.ds(i * L, L)] = zero
        return 0

    lax.fori_loop(0, nchunks, body, 0)


def _sc_body(x_hbm, mask_hbm, s_hbm, xb, keyb, sb, hist):
    wid = lax.axis_index("s") * NC + lax.axis_index("c")
    ones_i = jnp.ones((L,), jnp.int32)
    one_f = jnp.ones((L,), jnp.float32)
    zero_f = jnp.zeros((L,), jnp.float32)
    inv_temp = jnp.float32(1.0 / TEMPERATURE)

    for rr in range(2):
        r = wid * 2 + rr
        pltpu.sync_copy(x_hbm.at[r], xb)
        _zero_hist(hist, HB // L)

        def p1(i, _):
            v = xb[pl.ds(i * L, L)]
            bits = lax.bitcast_convert_type(v, jnp.int32)
            key_i = jnp.where(bits < 0, bits ^ jnp.int32(0x7FFFFFFF), bits)
            key_u = key_i ^ MIN32
            keyb[pl.ds(i * L, L)] = key_u
            y = (v + 1.0) * inv_temp
            sb[pl.ds(i * L, L)] = 1.0 / (1.0 + jnp.exp(-y))
            b = lax.shift_right_logical(key_u, 20)
            plsc.addupdate_scatter(hist, [b], ones_i)
            return 0

        lax.fori_loop(0, NV, p1, 0)
        b1, above1 = _find_bucket(hist, HB // L, jnp.int32(K))
        kk2 = jnp.int32(K) - above1

        _zero_hist(hist, HB // L)
        b1v = jnp.full((L,), b1, jnp.int32)

        def p2(i, _):
            ku = keyb[pl.ds(i * L, L)]
            top = lax.shift_right_logical(ku, 20)
            mid = jnp.bitwise_and(lax.shift_right_logical(ku, 8),
                                  jnp.int32(0xFFF))
            plsc.addupdate_scatter(hist, [mid], ones_i, mask=top == b1v)
            return 0

        lax.fori_loop(0, NV, p2, 0)
        b2, above2 = _find_bucket(hist, HB // L, kk2)
        kk3 = kk2 - above2

        _zero_hist(hist, 256 // L)
        pref = jnp.bitwise_or(lax.shift_left(b1, 12), b2)
        prefv = jnp.full((L,), pref, jnp.int32)

        def p3(i, _):
            ku = keyb[pl.ds(i * L, L)]
            hi = lax.shift_right_logical(ku, 8)
            low = jnp.bitwise_and(ku, jnp.int32(0xFF))
            plsc.addupdate_scatter(hist, [low], ones_i, mask=hi == prefv)
            return 0

        lax.fori_loop(0, NV, p3, 0)
        b3, _ = _find_bucket(hist, 256 // L, kk3)

        t_u = jnp.bitwise_or(lax.shift_left(b1, 20),
                             jnp.bitwise_or(lax.shift_left(b2, 8), b3))
        t_i = t_u ^ MIN32
        tv = jnp.full((L,), t_i, jnp.int32)

        def p4(i, _):
            ki = keyb[pl.ds(i * L, L)] ^ MIN32
            xb[pl.ds(i * L, L)] = jnp.where(ki >= tv, one_f, zero_f)
            return 0

        lax.fori_loop(0, NV, p4, 0)
        pltpu.sync_copy(xb, mask_hbm.at[r])
        pltpu.sync_copy(sb, s_hbm.at[r])


@jax.jit
def kernel(x):
    mesh = plsc.VectorSubcoreMesh(core_axis_name="c", subcore_axis_name="s")
    out = pl.kernel(
        _sc_body,
        out_type=(
            jax.ShapeDtypeStruct((ROWS, COLS), jnp.float32),
            jax.ShapeDtypeStruct((ROWS, COLS), jnp.float32),
        ),
        mesh=mesh,
        compiler_params=pltpu.CompilerParams(needs_layout_passes=False),
        scratch_types=[
            pltpu.VMEM((COLS,), jnp.float32),
            pltpu.VMEM((COLS,), jnp.int32),
            pltpu.VMEM((COLS,), jnp.float32),
            pltpu.VMEM((HB,), jnp.int32),
        ],
    )(x)
    return out
